# baseline (speedup 1.0000x reference)
import jax
import jax.numpy as jnp
from jax import lax
from jax.experimental import pallas as pl
from jax.experimental.pallas import tpu as pltpu
from jax.experimental.pallas import tpu_sc as plsc

NC = 2
NS = 16
L = 16
NW = NC * NS

VOCAB = 1000000
BATCH = 16384
D = 32
NR = 4
B_PER_W = BATCH // NW
IDX_CHUNK = 128
N_CHUNKS = B_PER_W // IDX_CHUNK
N_GROUPS = B_PER_W // L


def _body(u_idx_hbm, i_idx_hbm, qp_hbm, qs_hbm, qi_hbm, out_hbm,
          uidx_v, iidx_v, ridx_u, ridx_i, pv, sv, iv, out_v, sem):
    wid = lax.axis_index("s") * NC + lax.axis_index("c")
    base = wid * B_PER_W

    row0 = wid * N_CHUNKS
    pltpu.sync_copy(u_idx_hbm.at[pl.ds(row0, N_CHUNKS)], uidx_v)
    pltpu.sync_copy(i_idx_hbm.at[pl.ds(row0, N_CHUNKS)], iidx_v)

    for r in range(NR):
        for j in range(N_CHUNKS):
            for c in range(IDX_CHUNK // L):
                sl = pl.ds(c * L, L)
                ridx_u[r * N_CHUNKS + j, sl] = uidx_v[j, sl] + r * VOCAB
                ridx_i[r * N_CHUNKS + j, sl] = iidx_v[j, sl] + r * VOCAB

    copies = []
    for r in range(NR):
        for j in range(N_CHUNKS):
            k = r * N_CHUNKS + j
            dst = pl.ds(r * B_PER_W + j * IDX_CHUNK, IDX_CHUNK)
            copies.append(pltpu.async_copy(qp_hbm.at[ridx_u.at[k]],
                                           pv.at[dst], sem))
            copies.append(pltpu.async_copy(qs_hbm.at[ridx_u.at[k]],
                                           sv.at[dst], sem))
            copies.append(pltpu.async_copy(qi_hbm.at[ridx_i.at[k]],
                                           iv.at[dst], sem))
    for c in copies:
        c.wait()

    def group(g, carry):
        b0 = g * L
        acc = jnp.zeros((L,), jnp.float32)
        for d in range(D):
            r, s = divmod(d, 8)
            rows = r * B_PER_W + b0 + lax.iota(jnp.int32, L)
            col = jnp.full((L,), s, jnp.int32)
            p = plsc.load_gather(pv, [rows, col])
            q = plsc.load_gather(sv, [rows, col])
            t = plsc.load_gather(iv, [rows, col])
            acc = acc + (p + q) * t
        acc = acc * 0.5
        preds = 1.0 / (1.0 + jnp.exp(-acc))
        out_v[pl.ds(b0, L)] = preds
        return carry

    lax.fori_loop(0, N_GROUPS, group, 0)

    pltpu.sync_copy(out_v, out_hbm.at[pl.ds(base, B_PER_W)])


@jax.jit
def _run(u_idx2d, i_idx2d, qp, qs, qi):
    mesh = plsc.VectorSubcoreMesh(core_axis_name="c", subcore_axis_name="s")
    f = pl.kernel(
        _body,
        out_type=jax.ShapeDtypeStruct((BATCH,), jnp.float32),
        mesh=mesh,
        scratch_types=[
            pltpu.VMEM((N_CHUNKS, IDX_CHUNK), jnp.int32),
            pltpu.VMEM((N_CHUNKS, IDX_CHUNK), jnp.int32),
            pltpu.VMEM((NR * N_CHUNKS, IDX_CHUNK), jnp.int32),
            pltpu.VMEM((NR * N_CHUNKS, IDX_CHUNK), jnp.int32),
            pltpu.VMEM((NR * B_PER_W, 8), jnp.float32),
            pltpu.VMEM((NR * B_PER_W, 8), jnp.float32),
            pltpu.VMEM((NR * B_PER_W, 8), jnp.float32),
            pltpu.VMEM((B_PER_W,), jnp.float32),
            pltpu.SemaphoreType.DMA,
        ],
        compiler_params=pltpu.CompilerParams(needs_layout_passes=False,
                                             use_tc_tiling_on_sc=False),
    )
    return f(u_idx2d, i_idx2d, qp, qs, qi)


def _qform(table):
    return (table.T.reshape(NR, 8, VOCAB)
            .transpose(0, 2, 1)
            .reshape(NR * VOCAB, 8))


def kernel(user_inputs, u_item_inputs, user_table_private, user_table_shared,
           item_table):
    u2d = user_inputs.reshape(BATCH // IDX_CHUNK, IDX_CHUNK)
    i2d = u_item_inputs.reshape(BATCH // IDX_CHUNK, IDX_CHUNK)
    return _run(u2d, i2d, _qform(user_table_private),
                _qform(user_table_shared), _qform(item_table))


# speedup vs baseline: 3.5778x; 3.5778x over previous
import jax
import jax.numpy as jnp
from jax import lax
from jax.experimental import pallas as pl
from jax.experimental.pallas import tpu as pltpu
from jax.experimental.pallas import tpu_sc as plsc

NC = 2
NS = 16
L = 16
NW = NC * NS

BATCH = 16384
D = 32
B_PER_W = BATCH // NW
IDX_CHUNK = 128
N_CHUNKS = B_PER_W // IDX_CHUNK
N_GROUPS = B_PER_W // L


def _body(u_idx_hbm, i_idx_hbm, priv_hbm, shar_hbm, item_hbm, out_hbm,
          uidx_v, iidx_v, priv_v, shar_v, item_v, out_v, sem):
    wid = lax.axis_index("s") * NC + lax.axis_index("c")
    base = wid * B_PER_W

    row0 = wid * N_CHUNKS
    pltpu.sync_copy(u_idx_hbm.at[pl.ds(row0, N_CHUNKS)], uidx_v)
    pltpu.sync_copy(i_idx_hbm.at[pl.ds(row0, N_CHUNKS)], iidx_v)

    copies = []
    for j in range(N_CHUNKS):
        dst = pl.ds(j * IDX_CHUNK, IDX_CHUNK)
        copies.append(pltpu.async_copy(priv_hbm.at[uidx_v.at[j]],
                                       priv_v.at[dst], sem))
        copies.append(pltpu.async_copy(shar_hbm.at[uidx_v.at[j]],
                                       shar_v.at[dst], sem))
        copies.append(pltpu.async_copy(item_hbm.at[iidx_v.at[j]],
                                       item_v.at[dst], sem))
    for c in copies:
        c.wait()

    def group(g, carry):
        b0 = g * L
        rows = b0 + lax.iota(jnp.int32, L)
        acc = jnp.zeros((L,), jnp.float32)
        for j in range(D):
            col = jnp.full((L,), j, jnp.int32)
            p = plsc.load_gather(priv_v, [rows, col])
            s = plsc.load_gather(shar_v, [rows, col])
            it = plsc.load_gather(item_v, [rows, col])
            acc = acc + (p + s) * it
        acc = acc * 0.5
        preds = 1.0 / (1.0 + jnp.exp(-acc))
        out_v[pl.ds(b0, L)] = preds
        return carry

    lax.fori_loop(0, N_GROUPS, group, 0)

    pltpu.sync_copy(out_v, out_hbm.at[pl.ds(base, B_PER_W)])


@jax.jit
def _run(u_idx2d, i_idx2d, priv, shar, item):
    mesh = plsc.VectorSubcoreMesh(core_axis_name="c", subcore_axis_name="s")
    f = pl.kernel(
        _body,
        out_type=jax.ShapeDtypeStruct((BATCH,), jnp.float32),
        mesh=mesh,
        scratch_types=[
            pltpu.VMEM((N_CHUNKS, IDX_CHUNK), jnp.int32),
            pltpu.VMEM((N_CHUNKS, IDX_CHUNK), jnp.int32),
            pltpu.VMEM((B_PER_W, D), jnp.float32),
            pltpu.VMEM((B_PER_W, D), jnp.float32),
            pltpu.VMEM((B_PER_W, D), jnp.float32),
            pltpu.VMEM((B_PER_W,), jnp.float32),
            pltpu.SemaphoreType.DMA,
        ],
        compiler_params=pltpu.CompilerParams(needs_layout_passes=False,
                                             use_tc_tiling_on_sc=False),
    )
    return f(u_idx2d, i_idx2d, priv, shar, item)


def kernel(user_inputs, u_item_inputs, user_table_private, user_table_shared,
           item_table):
    u2d = user_inputs.reshape(BATCH // IDX_CHUNK, IDX_CHUNK)
    i2d = u_item_inputs.reshape(BATCH // IDX_CHUNK, IDX_CHUNK)
    return _run(u2d, i2d, user_table_private, user_table_shared, item_table)


# speedup vs baseline: 13.4900x; 3.7705x over previous
import jax
import jax.numpy as jnp
from jax import lax
from jax.experimental import pallas as pl
from jax.experimental.pallas import tpu as pltpu
from jax.experimental.pallas import tpu_sc as plsc

NC = 2
NS = 16
L = 16
NW = NC * NS

BATCH = 16384
D = 32
B_PER_W = BATCH // NW
CHUNK = 8
N_CHUNKS = B_PER_W // CHUNK
TILE_H = 8
TILE_W = 128
A = D // TILE_H


def _body(u_idx_hbm, i_idx_hbm, privT, sharT, itemT, out_hbm,
          uidx_v, iidx_v, dot_v, slots, sem):
    wid = lax.axis_index("s") * NC + lax.axis_index("c")
    base = wid * B_PER_W

    pltpu.sync_copy(u_idx_hbm.at[pl.ds(base, B_PER_W)], uidx_v)
    pltpu.sync_copy(i_idx_hbm.at[pl.ds(base, B_PER_W)], iidx_v)

    iota = lax.iota(jnp.int32, L)
    a_lo = iota // TILE_H
    a_hi = a_lo + 2
    sub = iota % TILE_H

    def group(g, carry):
        g0 = g * L
        uvec = uidx_v[pl.ds(g0, L)]
        ivec = iidx_v[pl.ds(g0, L)]
        dots = jnp.zeros((L,), jnp.float32)
        for half in range(L // CHUNK):
            lanes_u, lanes_i, copies = [], [], []
            for e in range(CHUNK):
                lane = half * CHUNK + e
                ur = uvec[lane]
                ir = ivec[lane]
                cu = pl.multiple_of((ur // TILE_W) * TILE_W, TILE_W)
                ci = pl.multiple_of((ir // TILE_W) * TILE_W, TILE_W)
                lanes_u.append(ur % TILE_W)
                lanes_i.append(ir % TILE_W)
                slot = slots.at[e]
                for a in range(A):
                    rows = pl.ds(a * TILE_H, TILE_H)
                    copies.append(pltpu.async_copy(
                        privT.at[rows, pl.ds(cu, TILE_W)],
                        slot.at[0].at[a], sem))
                    copies.append(pltpu.async_copy(
                        sharT.at[rows, pl.ds(cu, TILE_W)],
                        slot.at[1].at[a], sem))
                    copies.append(pltpu.async_copy(
                        itemT.at[rows, pl.ds(ci, TILE_W)],
                        slot.at[2].at[a], sem))
            for c in copies:
                c.wait()
            for e in range(CHUNK):
                lane = half * CHUNK + e
                jv_u = jnp.full((L,), lanes_u[e], jnp.int32)
                jv_i = jnp.full((L,), lanes_i[e], jnp.int32)
                slot = slots.at[e]
                p_lo = plsc.load_gather(slot.at[0], [a_lo, sub, jv_u])
                p_hi = plsc.load_gather(slot.at[0], [a_hi, sub, jv_u])
                s_lo = plsc.load_gather(slot.at[1], [a_lo, sub, jv_u])
                s_hi = plsc.load_gather(slot.at[1], [a_hi, sub, jv_u])
                t_lo = plsc.load_gather(slot.at[2], [a_lo, sub, jv_i])
                t_hi = plsc.load_gather(slot.at[2], [a_hi, sub, jv_i])
                acc = (p_lo + s_lo) * t_lo + (p_hi + s_hi) * t_hi
                dots = jnp.where(iota == lane, 0.5 * jnp.sum(acc), dots)
        dot_v[pl.ds(g0, L)] = 1.0 / (1.0 + jnp.exp(-dots))
        return carry

    lax.fori_loop(0, B_PER_W // L, group, 0)

    pltpu.sync_copy(dot_v, out_hbm.at[pl.ds(base, B_PER_W)])


@jax.jit
def _run(u_idx, i_idx, priv, shar, item):
    mesh = plsc.VectorSubcoreMesh(core_axis_name="c", subcore_axis_name="s")
    f = pl.kernel(
        _body,
        out_type=jax.ShapeDtypeStruct((BATCH,), jnp.float32),
        mesh=mesh,
        scratch_types=[
            pltpu.VMEM((B_PER_W,), jnp.int32),
            pltpu.VMEM((B_PER_W,), jnp.int32),
            pltpu.VMEM((B_PER_W,), jnp.float32),
            pltpu.VMEM((CHUNK, 3, A, TILE_H, TILE_W), jnp.float32),
            pltpu.SemaphoreType.DMA,
        ],
        compiler_params=pltpu.CompilerParams(needs_layout_passes=False,
                                             use_tc_tiling_on_sc=True),
    )
    return f(u_idx, i_idx, priv.T, shar.T, item.T)


def kernel(user_inputs, u_item_inputs, user_table_private, user_table_shared,
           item_table):
    return _run(user_inputs, u_item_inputs, user_table_private,
                user_table_shared, item_table)


# speedup vs baseline: 13.5718x; 1.0061x over previous
import jax
import jax.numpy as jnp
from jax import lax
from jax.experimental import pallas as pl
from jax.experimental.pallas import tpu as pltpu
from jax.experimental.pallas import tpu_sc as plsc

NC = 2
NS = 16
L = 16
NW = NC * NS

BATCH = 16384
D = 32
B_PER_W = BATCH // NW
CHUNK = 8
N_CHUNKS = B_PER_W // CHUNK
TILE_H = 8
TILE_W = 128
SLICE_W = 16
A = D // TILE_H


def _body(u_idx_hbm, i_idx_hbm, privT, sharT, itemT, out_hbm,
          uidx_v, iidx_v, dot_v, slots, sem):
    wid = lax.axis_index("s") * NC + lax.axis_index("c")
    base = wid * B_PER_W

    pltpu.sync_copy(u_idx_hbm.at[pl.ds(base, B_PER_W)], uidx_v)
    pltpu.sync_copy(i_idx_hbm.at[pl.ds(base, B_PER_W)], iidx_v)

    iota = lax.iota(jnp.int32, L)
    d_lo = iota
    d_hi = iota + L

    def group(g, carry):
        g0 = g * L
        uvec = uidx_v[pl.ds(g0, L)]
        ivec = iidx_v[pl.ds(g0, L)]
        dots = jnp.zeros((L,), jnp.float32)
        for half in range(L // CHUNK):
            lanes_u, lanes_i, copies = [], [], []
            for e in range(CHUNK):
                lane = half * CHUNK + e
                ur = uvec[lane]
                ir = ivec[lane]
                cu = pl.multiple_of((ur // TILE_W) * TILE_W, TILE_W)
                ci = pl.multiple_of((ir // TILE_W) * TILE_W, TILE_W)
                lanes_u.append(ur % TILE_W)
                lanes_i.append(ir % TILE_W)
                slot = slots.at[e]
                copies.append(pltpu.async_copy(
                    privT.at[:, pl.ds(cu, TILE_W)], slot.at[0], sem))
                copies.append(pltpu.async_copy(
                    sharT.at[:, pl.ds(cu, TILE_W)], slot.at[1], sem))
                copies.append(pltpu.async_copy(
                    itemT.at[:, pl.ds(ci, TILE_W)], slot.at[2], sem))
            for c in copies:
                c.wait()
            for e in range(CHUNK):
                lane = half * CHUNK + e
                jv_u = jnp.full((L,), lanes_u[e], jnp.int32)
                jv_i = jnp.full((L,), lanes_i[e], jnp.int32)
                slot = slots.at[e]
                p_lo = plsc.load_gather(slot.at[0], [d_lo, jv_u])
                p_hi = plsc.load_gather(slot.at[0], [d_hi, jv_u])
                s_lo = plsc.load_gather(slot.at[1], [d_lo, jv_u])
                s_hi = plsc.load_gather(slot.at[1], [d_hi, jv_u])
                t_lo = plsc.load_gather(slot.at[2], [d_lo, jv_i])
                t_hi = plsc.load_gather(slot.at[2], [d_hi, jv_i])
                acc = (p_lo + s_lo) * t_lo + (p_hi + s_hi) * t_hi
                dots = jnp.where(iota == lane, 0.5 * jnp.sum(acc), dots)
        dot_v[pl.ds(g0, L)] = 1.0 / (1.0 + jnp.exp(-dots))
        return carry

    lax.fori_loop(0, B_PER_W // L, group, 0)

    pltpu.sync_copy(dot_v, out_hbm.at[pl.ds(base, B_PER_W)])


@jax.jit
def _run(u_idx, i_idx, priv, shar, item):
    mesh = plsc.VectorSubcoreMesh(core_axis_name="c", subcore_axis_name="s")
    f = pl.kernel(
        _body,
        out_type=jax.ShapeDtypeStruct((BATCH,), jnp.float32),
        mesh=mesh,
        scratch_types=[
            pltpu.VMEM((B_PER_W,), jnp.int32),
            pltpu.VMEM((B_PER_W,), jnp.int32),
            pltpu.VMEM((B_PER_W,), jnp.float32),
            pltpu.VMEM((CHUNK, 3, D, TILE_W), jnp.float32),
            pltpu.SemaphoreType.DMA,
        ],
        compiler_params=pltpu.CompilerParams(needs_layout_passes=False,
                                             use_tc_tiling_on_sc=True),
    )
    return f(u_idx, i_idx, priv.T, shar.T, item.T)


def kernel(user_inputs, u_item_inputs, user_table_private, user_table_shared,
           item_table):
    return _run(user_inputs, u_item_inputs, user_table_private,
                user_table_shared, item_table)


# speedup vs baseline: 15.6256x; 1.1513x over previous
import jax
import jax.numpy as jnp
from jax import lax
from jax.experimental import pallas as pl
from jax.experimental.pallas import tpu as pltpu
from jax.experimental.pallas import tpu_sc as plsc

NC = 2
NS = 16
L = 16
NW = NC * NS

BATCH = 16384
D = 32
B_PER_W = BATCH // NW
RING = 8
TILE_W = 128
N_GROUPS = B_PER_W // L


def _body(u_idx_hbm, i_idx_hbm, privT, sharT, itemT, out_hbm,
          uidx_v, iidx_v, dot_v, slots, sems):
    wid = lax.axis_index("s") * NC + lax.axis_index("c")
    base = wid * B_PER_W

    pltpu.sync_copy(u_idx_hbm.at[pl.ds(base, B_PER_W)], uidx_v)
    pltpu.sync_copy(i_idx_hbm.at[pl.ds(base, B_PER_W)], iidx_v)

    iota = lax.iota(jnp.int32, L)
    d_lo = iota
    d_hi = iota + L

    def issue(slot_e, sem_e, ur, ir):
        cu = pl.multiple_of((ur // TILE_W) * TILE_W, TILE_W)
        ci = pl.multiple_of((ir // TILE_W) * TILE_W, TILE_W)
        pltpu.async_copy(privT.at[:, pl.ds(cu, TILE_W)], slot_e.at[0], sem_e)
        pltpu.async_copy(sharT.at[:, pl.ds(cu, TILE_W)], slot_e.at[1], sem_e)
        pltpu.async_copy(itemT.at[:, pl.ds(ci, TILE_W)], slot_e.at[2], sem_e)

    def drain(slot_e, sem_e):
        for t in range(3):
            pltpu.make_async_copy(
                privT.at[:, pl.ds(0, TILE_W)], slot_e.at[t], sem_e).wait()

    def compute(slot_e, ju, ji):
        jv_u = jnp.full((L,), ju, jnp.int32)
        jv_i = jnp.full((L,), ji, jnp.int32)
        p_lo = plsc.load_gather(slot_e.at[0], [d_lo, jv_u])
        p_hi = plsc.load_gather(slot_e.at[0], [d_hi, jv_u])
        s_lo = plsc.load_gather(slot_e.at[1], [d_lo, jv_u])
        s_hi = plsc.load_gather(slot_e.at[1], [d_hi, jv_u])
        t_lo = plsc.load_gather(slot_e.at[2], [d_lo, jv_i])
        t_hi = plsc.load_gather(slot_e.at[2], [d_hi, jv_i])
        acc = (p_lo + s_lo) * t_lo + (p_hi + s_hi) * t_hi
        return 0.5 * jnp.sum(acc)

    uvec0 = uidx_v[pl.ds(0, L)]
    ivec0 = iidx_v[pl.ds(0, L)]
    for e in range(RING):
        issue(slots.at[e], sems.at[e], uvec0[e], ivec0[e])

    def group(g, carry):
        g0 = g * L
        uvec = uidx_v[pl.ds(g0, L)]
        ivec = iidx_v[pl.ds(g0, L)]
        gn = jnp.minimum(g0 + L, B_PER_W - L)
        uvecN = uidx_v[pl.ds(gn, L)]
        ivecN = iidx_v[pl.ds(gn, L)]
        dots = jnp.zeros((L,), jnp.float32)
        for e in range(RING):
            lane = RING + e
            drain(slots.at[e], sems.at[e])
            dots = jnp.where(iota == e,
                             compute(slots.at[e], uvec[e] % TILE_W,
                                     ivec[e] % TILE_W), dots)
            issue(slots.at[e], sems.at[e], uvec[lane], ivec[lane])
        for e in range(RING):
            lane = RING + e
            drain(slots.at[e], sems.at[e])
            dots = jnp.where(iota == lane,
                             compute(slots.at[e], uvec[lane] % TILE_W,
                                     ivec[lane] % TILE_W), dots)
            issue(slots.at[e], sems.at[e], uvecN[e], ivecN[e])
        dot_v[pl.ds(g0, L)] = 1.0 / (1.0 + jnp.exp(-dots))
        return carry

    lax.fori_loop(0, N_GROUPS, group, 0)

    for e in range(RING):
        drain(slots.at[e], sems.at[e])

    pltpu.sync_copy(dot_v, out_hbm.at[pl.ds(base, B_PER_W)])


@jax.jit
def _run(u_idx, i_idx, priv, shar, item):
    mesh = plsc.VectorSubcoreMesh(core_axis_name="c", subcore_axis_name="s")
    f = pl.kernel(
        _body,
        out_type=jax.ShapeDtypeStruct((BATCH,), jnp.float32),
        mesh=mesh,
        scratch_types=[
            pltpu.VMEM((B_PER_W,), jnp.int32),
            pltpu.VMEM((B_PER_W,), jnp.int32),
            pltpu.VMEM((B_PER_W,), jnp.float32),
            pltpu.VMEM((RING, 3, D, TILE_W), jnp.float32),
            pltpu.SemaphoreType.DMA((RING,)),
        ],
        compiler_params=pltpu.CompilerParams(needs_layout_passes=False,
                                             use_tc_tiling_on_sc=True),
    )
    return f(u_idx, i_idx, priv.T, shar.T, item.T)


def kernel(user_inputs, u_item_inputs, user_table_private, user_table_shared,
           item_table):
    return _run(user_inputs, u_item_inputs, user_table_private,
                user_table_shared, item_table)
